# baseline (speedup 1.0000x reference)
import jax
import jax.numpy as jnp
from jax import lax
from jax.experimental import pallas as pl
from jax.experimental.pallas import tpu as pltpu
from jax.experimental.pallas import tpu_sc as plsc

E = 64
H = 1024
I = 512
T = 2048
B = 32
G = T // B + E
R = G * B

CHUNK_BLOCKS = (16, 48, 48, 16)
assert sum(CHUNK_BLOCKS) == G

_SC_INFO = plsc.get_sparse_core_info()
NW = _SC_INFO.num_cores * _SC_INFO.num_subcores


def _sc_mesh():
    return plsc.VectorSubcoreMesh(core_axis_name="c", subcore_axis_name="s")


def _wid():
    return lax.axis_index("s") * _SC_INFO.num_cores + lax.axis_index("c")


def _make_gather(rows):
    rpw = rows // NW

    def body(hs_hbm, idx_hbm, out_hbm, idx_v, buf, sem):
        base = _wid() * rpw
        pltpu.sync_copy(idx_hbm.at[pl.ds(base, rpw)], idx_v)
        pltpu.async_copy(hs_hbm.at[idx_v], buf, sem).wait()
        pltpu.sync_copy(buf, out_hbm.at[pl.ds(base, rpw)])

    return pl.kernel(
        body,
        mesh=_sc_mesh(),
        out_type=jax.ShapeDtypeStruct((rows, H), jnp.float32),
        scratch_types=[
            pltpu.VMEM((rpw,), jnp.int32),
            pltpu.VMEM((rpw, H), jnp.float32),
            pltpu.SemaphoreType.DMA,
        ],
    )


def _make_scatter(rows):
    rpw = rows // NW

    def body(y_hbm, idx_hbm, out_hbm, idx_v, buf, sem):
        base = _wid() * rpw
        pltpu.sync_copy(idx_hbm.at[pl.ds(base, rpw)], idx_v)
        pltpu.sync_copy(y_hbm.at[pl.ds(base, rpw)], buf)
        pltpu.async_copy(buf, out_hbm.at[idx_v], sem).wait()

    return pl.kernel(
        body,
        mesh=_sc_mesh(),
        out_type=(),
        scratch_types=[
            pltpu.VMEM((rpw,), jnp.int32),
            pltpu.VMEM((rpw, H), jnp.float32),
            pltpu.SemaphoreType.DMA,
        ],
    )


def _mm_body(bte_ref, x_ref, w_ref, gu_ref, dp_ref, o_ref):
    x = x_ref[...]
    gu = lax.dot_general(
        x, gu_ref[0],
        (((1,), (1,)), ((), ())),
        preferred_element_type=jnp.float32,
    )
    gate = gu[:, :I]
    up = gu[:, I:]
    act = gate * jax.nn.sigmoid(gate) * up
    y = lax.dot_general(
        act, dp_ref[0],
        (((1,), (1,)), ((), ())),
        preferred_element_type=jnp.float32,
    )
    o_ref[...] = y * w_ref[0, 0][:, None]


def _tc_grouped_matmul(nblk, x_chunk, w_pad, gate_up_proj, down_proj, bte):
    grid_spec = pltpu.PrefetchScalarGridSpec(
        num_scalar_prefetch=1,
        grid=(nblk,),
        in_specs=[
            pl.BlockSpec((B, H), lambda g, bte: (g, 0)),
            pl.BlockSpec((1, 1, B), lambda g, bte: (g, 0, 0)),
            pl.BlockSpec((1, 2 * I, H), lambda g, bte: (bte[g], 0, 0)),
            pl.BlockSpec((1, H, I), lambda g, bte: (bte[g], 0, 0)),
        ],
        out_specs=pl.BlockSpec((B, H), lambda g, bte: (g, 0)),
    )
    return pl.pallas_call(
        _mm_body,
        grid_spec=grid_spec,
        out_shape=jax.ShapeDtypeStruct((nblk * B, H), jnp.float32),
    )(bte, x_chunk, w_pad, gate_up_proj, down_proj)


@jax.jit
def kernel(hidden_states, top_k_index, top_k_weights, gate_up_proj, down_proj):
    e = top_k_index[:, 0].astype(jnp.int32)
    w = top_k_weights[:, 0]

    order = jnp.argsort(e).astype(jnp.int32)
    e_sorted = e[order]

    counts = jnp.bincount(e, length=E)
    offsets = jnp.cumsum(counts) - counts
    blocks_per_e = (counts + B - 1) // B
    blocks_end = jnp.cumsum(blocks_per_e)
    blocks_start = blocks_end - blocks_per_e

    pos = jnp.arange(T, dtype=jnp.int32)
    slot = blocks_start[e_sorted] * B + pos - offsets[e_sorted]

    gather_ids = jnp.zeros((R,), jnp.int32).at[slot].set(order)
    scatter_ids = jnp.full((R,), T, jnp.int32).at[slot].set(order)
    w_pad = jnp.zeros((G, 1, B), jnp.float32).at[
        slot // B, 0, slot % B].set(w[order])

    bte = jnp.searchsorted(
        blocks_end, jnp.arange(G, dtype=jnp.int32), side="right"
    ).astype(jnp.int32)
    bte = jnp.minimum(bte, E - 1)

    out_ref = jax.new_ref(jnp.zeros((T + 8, H), jnp.float32))

    b0 = 0
    for nblk in CHUNK_BLOCKS:
        rows = nblk * B
        r0 = b0 * B
        x_c = _make_gather(rows)(
            hidden_states, lax.dynamic_slice_in_dim(gather_ids, r0, rows))
        y_c = _tc_grouped_matmul(
            nblk, x_c,
            lax.dynamic_slice_in_dim(w_pad, b0, nblk),
            gate_up_proj, down_proj,
            lax.dynamic_slice_in_dim(bte, b0, nblk))
        _make_scatter(rows)(
            y_c, lax.dynamic_slice_in_dim(scatter_ids, r0, rows), out_ref)
        b0 += nblk

    return out_ref[...][:T]


# speedup vs baseline: 1.0052x; 1.0052x over previous
import jax
import jax.numpy as jnp
from jax import lax
from jax.experimental import pallas as pl
from jax.experimental.pallas import tpu as pltpu
from jax.experimental.pallas import tpu_sc as plsc

E = 64
H = 1024
I = 512
T = 2048
B = 32
G = T // B + E
R = G * B

CHUNK_BLOCKS = (16, 48, 48, 16)
assert sum(CHUNK_BLOCKS) == G

_SC_INFO = plsc.get_sparse_core_info()
NW = _SC_INFO.num_cores * _SC_INFO.num_subcores


def _sc_mesh():
    return plsc.VectorSubcoreMesh(core_axis_name="c", subcore_axis_name="s")


def _wid():
    return lax.axis_index("s") * _SC_INFO.num_cores + lax.axis_index("c")


def _make_gather(rows):
    rpw = rows // NW

    def body(hs_hbm, idx_hbm, out_hbm, idx_v, buf, sem):
        base = _wid() * rpw
        pltpu.sync_copy(idx_hbm.at[pl.ds(base, rpw)], idx_v)
        pltpu.async_copy(hs_hbm.at[idx_v], buf, sem).wait()
        pltpu.sync_copy(buf, out_hbm.at[pl.ds(base, rpw)])

    return pl.kernel(
        body,
        mesh=_sc_mesh(),
        out_type=jax.ShapeDtypeStruct((rows, H), jnp.float32),
        scratch_types=[
            pltpu.VMEM((rpw,), jnp.int32),
            pltpu.VMEM((rpw, H), jnp.float32),
            pltpu.SemaphoreType.DMA,
        ],
    )


def _make_scatter(rows):
    rpw = rows // NW

    def body(y_hbm, idx_hbm, out_hbm, idx_v, buf, sem):
        base = _wid() * rpw
        pltpu.sync_copy(idx_hbm.at[pl.ds(base, rpw)], idx_v)
        pltpu.sync_copy(y_hbm.at[pl.ds(base, rpw)], buf)
        pltpu.async_copy(buf, out_hbm.at[idx_v], sem).wait()

    return pl.kernel(
        body,
        mesh=_sc_mesh(),
        out_type=(),
        scratch_types=[
            pltpu.VMEM((rpw,), jnp.int32),
            pltpu.VMEM((rpw, H), jnp.float32),
            pltpu.SemaphoreType.DMA,
        ],
    )


def _mm_body(bte_ref, x_ref, w_ref, gate_ref, up_ref, dp0_ref, dp1_ref, o_ref):
    x = x_ref[...]
    gate = lax.dot_general(
        x, gate_ref[0],
        (((1,), (1,)), ((), ())),
        preferred_element_type=jnp.float32,
    )
    up = lax.dot_general(
        x, up_ref[0],
        (((1,), (1,)), ((), ())),
        preferred_element_type=jnp.float32,
    )
    act = gate * jax.nn.sigmoid(gate) * up
    y0 = lax.dot_general(
        act, dp0_ref[0],
        (((1,), (1,)), ((), ())),
        preferred_element_type=jnp.float32,
    )
    y1 = lax.dot_general(
        act, dp1_ref[0],
        (((1,), (1,)), ((), ())),
        preferred_element_type=jnp.float32,
    )
    y = jnp.concatenate([y0, y1], axis=1)
    o_ref[...] = y * w_ref[0, 0][:, None]


def _tc_grouped_matmul(nblk, x_chunk, w_pad, gate_up_proj, down_proj, bte):
    grid_spec = pltpu.PrefetchScalarGridSpec(
        num_scalar_prefetch=1,
        grid=(nblk,),
        in_specs=[
            pl.BlockSpec((B, H), lambda g, bte: (g, 0)),
            pl.BlockSpec((1, 1, B), lambda g, bte: (g, 0, 0)),
            pl.BlockSpec((1, I, H), lambda g, bte: (bte[g], 0, 0)),
            pl.BlockSpec((1, I, H), lambda g, bte: (bte[g], 1, 0)),
            pl.BlockSpec((1, H // 2, I), lambda g, bte: (bte[g], 0, 0)),
            pl.BlockSpec((1, H // 2, I), lambda g, bte: (bte[g], 1, 0)),
        ],
        out_specs=pl.BlockSpec((B, H), lambda g, bte: (g, 0)),
    )
    return pl.pallas_call(
        _mm_body,
        grid_spec=grid_spec,
        out_shape=jax.ShapeDtypeStruct((nblk * B, H), jnp.float32),
    )(bte, x_chunk, w_pad, gate_up_proj, gate_up_proj, down_proj, down_proj)


@jax.jit
def kernel(hidden_states, top_k_index, top_k_weights, gate_up_proj, down_proj):
    e = top_k_index[:, 0].astype(jnp.int32)
    w = top_k_weights[:, 0]

    order = jnp.argsort(e).astype(jnp.int32)
    e_sorted = e[order]

    counts = jnp.bincount(e, length=E)
    offsets = jnp.cumsum(counts) - counts
    blocks_per_e = (counts + B - 1) // B
    blocks_end = jnp.cumsum(blocks_per_e)
    blocks_start = blocks_end - blocks_per_e

    pos = jnp.arange(T, dtype=jnp.int32)
    slot = blocks_start[e_sorted] * B + pos - offsets[e_sorted]

    gather_ids = jnp.zeros((R,), jnp.int32).at[slot].set(order)
    scatter_ids = jnp.full((R,), T, jnp.int32).at[slot].set(order)
    w_pad = jnp.zeros((G, 1, B), jnp.float32).at[
        slot // B, 0, slot % B].set(w[order])

    bte = jnp.searchsorted(
        blocks_end, jnp.arange(G, dtype=jnp.int32), side="right"
    ).astype(jnp.int32)
    bte = jnp.minimum(bte, E - 1)

    out_ref = jax.new_ref(jnp.zeros((T + 8, H), jnp.float32))

    b0 = 0
    for nblk in CHUNK_BLOCKS:
        rows = nblk * B
        r0 = b0 * B
        x_c = _make_gather(rows)(
            hidden_states, lax.dynamic_slice_in_dim(gather_ids, r0, rows))
        y_c = _tc_grouped_matmul(
            nblk, x_c,
            lax.dynamic_slice_in_dim(w_pad, b0, nblk),
            gate_up_proj, down_proj,
            lax.dynamic_slice_in_dim(bte, b0, nblk))
        _make_scatter(rows)(
            y_c, lax.dynamic_slice_in_dim(scatter_ids, r0, rows), out_ref)
        b0 += nblk

    return out_ref[...][:T]
